# baseline (speedup 1.0000x reference)
import functools

import jax
import jax.numpy as jnp
from jax import lax
from jax.experimental import pallas as pl
from jax.experimental.pallas import tpu as pltpu
from jax.experimental.pallas import tpu_sc as plsc

LANES = 16
CHUNK = 128
EMBW = 2
WIDE = 128
ROWS_PER_BLOCK = 16384


def _sigmoid(x):
    return 1.0 / (1.0 + jnp.exp(-x))


def _widen(table):
    n = table.shape[0]
    r = ROWS_PER_BLOCK

    def body(t_ref, y_ref):
        sel = (lax.broadcasted_iota(jnp.int32, (2, WIDE), 0) ==
               lax.broadcasted_iota(jnp.int32, (2, WIDE), 1)).astype(jnp.float32)
        y_ref[...] = jnp.dot(t_ref[...], sel,
                             preferred_element_type=jnp.float32)

    return pl.pallas_call(
        body,
        grid=((n + r - 1) // r,),
        in_specs=[pl.BlockSpec((r, 2), lambda i: (i, 0))],
        out_specs=pl.BlockSpec((r, WIDE), lambda i: (i, 0)),
        out_shape=jax.ShapeDtypeStruct((n, WIDE), jnp.float32),
    )(table)


def kernel(x_uid, x_mid, uid_table, mid_table):
    batch = x_uid.shape[0]
    info = plsc.get_sparse_core_info()
    nc, ns = info.num_cores, info.num_subcores
    nw = nc * ns
    bpw = batch // nw
    nchunk = bpw // CHUNK

    mesh = plsc.VectorSubcoreMesh(core_axis_name="c", subcore_axis_name="s")

    @functools.partial(
        pl.kernel,
        mesh=mesh,
        out_type=jax.ShapeDtypeStruct((batch,), jnp.float32),
        scratch_types=[
            pltpu.VMEM((bpw,), jnp.int32),
            pltpu.VMEM((bpw,), jnp.int32),
            pltpu.VMEM((bpw,), jnp.int32),
            pltpu.VMEM((bpw,), jnp.int32),
            pltpu.VMEM((bpw,), jnp.int32),
            pltpu.VMEM((bpw,), jnp.int32),
            pltpu.VMEM((bpw,), jnp.float32),
            pltpu.VMEM((bpw,), jnp.float32),
            pltpu.VMEM((bpw,), jnp.float32),
            pltpu.VMEM((bpw,), jnp.float32),
            pltpu.VMEM((bpw,), jnp.float32),
            pltpu.SemaphoreType.DMA,
        ],
    )
    def sc_kernel(xu_hbm, xm_hbm, ut_hbm, mt_hbm, out_hbm,
                  idx_u, idx_m, iu0, iu1, im0, im1,
                  u0v, u1v, m0v, m1v, out_v, sem):
        wid = lax.axis_index("s") * nc + lax.axis_index("c")
        base = wid * bpw

        cu = pltpu.async_copy(xu_hbm.at[pl.ds(base, bpw)], idx_u, sem)
        cm = pltpu.async_copy(xm_hbm.at[pl.ds(base, bpw)], idx_m, sem)
        cu.wait()
        cm.wait()

        for k in range(bpw // LANES):
            sl = pl.ds(k * LANES, LANES)
            eu = idx_u[sl] * EMBW
            em = idx_m[sl] * EMBW
            iu0[sl] = eu
            iu1[sl] = eu + 1
            im0[sl] = em
            im1[sl] = em + 1

        cps = []
        for j in range(nchunk):
            sl = pl.ds(j * CHUNK, CHUNK)
            cps.append(pltpu.async_copy(ut_hbm.at[iu0.at[sl]], u0v.at[sl], sem))
            cps.append(pltpu.async_copy(ut_hbm.at[iu1.at[sl]], u1v.at[sl], sem))
            cps.append(pltpu.async_copy(mt_hbm.at[im0.at[sl]], m0v.at[sl], sem))
            cps.append(pltpu.async_copy(mt_hbm.at[im1.at[sl]], m1v.at[sl], sem))
        for c in cps:
            c.wait()

        for k in range(bpw // LANES):
            sl = pl.ds(k * LANES, LANES)
            out_v[sl] = (_sigmoid(u0v[sl] - m0v[sl]) +
                         _sigmoid(u1v[sl] - m1v[sl])) * 0.5

        pltpu.sync_copy(out_v, out_hbm.at[pl.ds(base, bpw)])

    one = (x_uid[0] * 0 + 1).astype(jnp.float32)
    ut_flat = uid_table.reshape(-1) * one
    mt_flat = mid_table.reshape(-1) * one
    return sc_kernel(x_uid, x_mid, ut_flat, mt_flat)


# speedup vs baseline: 2.1357x; 2.1357x over previous
import functools

import jax
import jax.numpy as jnp
from jax import lax
from jax.experimental import pallas as pl
from jax.experimental.pallas import tpu as pltpu
from jax.experimental.pallas import tpu_sc as plsc

LANES = 16
CHUNK = 128
EMBW = 2
WIDE = 128
ROWS_PER_BLOCK = 8192


def _sigmoid(x):
    return 1.0 / (1.0 + jnp.exp(-x))


def _widen(table):
    n = table.shape[0]
    r = ROWS_PER_BLOCK

    def body(t_ref, y_ref):
        sel = (lax.broadcasted_iota(jnp.int32, (2, WIDE), 0) ==
               lax.broadcasted_iota(jnp.int32, (2, WIDE), 1)).astype(jnp.float32)
        y_ref[...] = jnp.dot(t_ref[...], sel,
                             preferred_element_type=jnp.float32)

    return pl.pallas_call(
        body,
        grid=((n + r - 1) // r,),
        in_specs=[pl.BlockSpec((r, 2), lambda i: (i, 0))],
        out_specs=pl.BlockSpec((r, WIDE), lambda i: (i, 0)),
        out_shape=jax.ShapeDtypeStruct((n, WIDE), jnp.float32),
    )(table)


def kernel(x_uid, x_mid, uid_table, mid_table):
    batch = x_uid.shape[0]
    info = plsc.get_sparse_core_info()
    nc, ns = info.num_cores, info.num_subcores
    nw = nc * ns
    bpw = batch // nw
    nchunk = bpw // CHUNK

    mesh = plsc.VectorSubcoreMesh(core_axis_name="c", subcore_axis_name="s")

    @functools.partial(
        pl.kernel,
        mesh=mesh,
        out_type=jax.ShapeDtypeStruct((batch,), jnp.float32),
        scratch_types=[
            pltpu.VMEM((bpw,), jnp.int32),
            pltpu.VMEM((bpw,), jnp.int32),
            pltpu.VMEM((bpw,), jnp.int32),
            pltpu.VMEM((bpw,), jnp.int32),
            pltpu.VMEM((bpw,), jnp.int32),
            pltpu.VMEM((bpw,), jnp.int32),
            pltpu.VMEM((bpw,), jnp.float32),
            pltpu.VMEM((bpw,), jnp.float32),
            pltpu.VMEM((bpw,), jnp.float32),
            pltpu.VMEM((bpw,), jnp.float32),
            pltpu.VMEM((bpw,), jnp.float32),
            pltpu.SemaphoreType.DMA,
        ],
    )
    def sc_kernel(xu_hbm, xm_hbm, ut_hbm, mt_hbm, out_hbm,
                  idx_u, idx_m, iu0, iu1, im0, im1,
                  u0v, u1v, m0v, m1v, out_v, sem):
        wid = lax.axis_index("s") * nc + lax.axis_index("c")
        base = wid * bpw

        cu = pltpu.async_copy(xu_hbm.at[pl.ds(base, bpw)], idx_u, sem)
        cm = pltpu.async_copy(xm_hbm.at[pl.ds(base, bpw)], idx_m, sem)
        cu.wait()
        cm.wait()

        for k in range(bpw // LANES):
            sl = pl.ds(k * LANES, LANES)
            eu = idx_u[sl] * WIDE
            em = idx_m[sl] * WIDE
            iu0[sl] = eu
            iu1[sl] = eu + 1
            im0[sl] = em
            im1[sl] = em + 1

        cps = []
        for j in range(nchunk):
            sl = pl.ds(j * CHUNK, CHUNK)
            cps.append(pltpu.async_copy(ut_hbm.at[iu0.at[sl]], u0v.at[sl], sem))
            cps.append(pltpu.async_copy(ut_hbm.at[iu1.at[sl]], u1v.at[sl], sem))
            cps.append(pltpu.async_copy(mt_hbm.at[im0.at[sl]], m0v.at[sl], sem))
            cps.append(pltpu.async_copy(mt_hbm.at[im1.at[sl]], m1v.at[sl], sem))
        for c in cps:
            c.wait()

        for k in range(bpw // LANES):
            sl = pl.ds(k * LANES, LANES)
            out_v[sl] = (_sigmoid(u0v[sl] - m0v[sl]) +
                         _sigmoid(u1v[sl] - m1v[sl])) * 0.5

        pltpu.sync_copy(out_v, out_hbm.at[pl.ds(base, bpw)])

    ut_wide = _widen(uid_table).reshape(-1)
    mt_wide = _widen(mid_table).reshape(-1)
    return sc_kernel(x_uid, x_mid, ut_wide, mt_wide)


# speedup vs baseline: 2.1535x; 1.0083x over previous
import functools

import jax
import jax.numpy as jnp
from jax import lax
from jax.experimental import pallas as pl
from jax.experimental.pallas import tpu as pltpu
from jax.experimental.pallas import tpu_sc as plsc

LANES = 16
CHUNK = 128
EMBW = 2
WIDE = 128
ROWS_PER_BLOCK = 16384


def _sigmoid(x):
    return 1.0 / (1.0 + jnp.exp(-x))


def _widen(table):
    n = table.shape[0]
    r = ROWS_PER_BLOCK

    def body(t_ref, y_ref):
        sel = (lax.broadcasted_iota(jnp.int32, (2, WIDE), 0) ==
               lax.broadcasted_iota(jnp.int32, (2, WIDE), 1)).astype(jnp.float32)
        y_ref[...] = jnp.dot(t_ref[...], sel,
                             preferred_element_type=jnp.float32)

    return pl.pallas_call(
        body,
        grid=((n + r - 1) // r,),
        in_specs=[pl.BlockSpec((r, 2), lambda i: (i, 0))],
        out_specs=pl.BlockSpec((r, WIDE), lambda i: (i, 0)),
        out_shape=jax.ShapeDtypeStruct((n, WIDE), jnp.float32),
    )(table)


def kernel(x_uid, x_mid, uid_table, mid_table):
    batch = x_uid.shape[0]
    info = plsc.get_sparse_core_info()
    nc, ns = info.num_cores, info.num_subcores
    nw = nc * ns
    bpw = batch // nw
    nchunk = bpw // CHUNK

    mesh = plsc.VectorSubcoreMesh(core_axis_name="c", subcore_axis_name="s")

    @functools.partial(
        pl.kernel,
        mesh=mesh,
        out_type=jax.ShapeDtypeStruct((batch,), jnp.float32),
        scratch_types=[
            pltpu.VMEM((bpw,), jnp.int32),
            pltpu.VMEM((bpw,), jnp.int32),
            pltpu.VMEM((bpw,), jnp.int32),
            pltpu.VMEM((bpw,), jnp.int32),
            pltpu.VMEM((bpw,), jnp.int32),
            pltpu.VMEM((bpw,), jnp.int32),
            pltpu.VMEM((bpw,), jnp.float32),
            pltpu.VMEM((bpw,), jnp.float32),
            pltpu.VMEM((bpw,), jnp.float32),
            pltpu.VMEM((bpw,), jnp.float32),
            pltpu.VMEM((bpw,), jnp.float32),
            pltpu.SemaphoreType.DMA,
        ],
    )
    def sc_kernel(xu_hbm, xm_hbm, ut_hbm, mt_hbm, out_hbm,
                  idx_u, idx_m, iu0, iu1, im0, im1,
                  u0v, u1v, m0v, m1v, out_v, sem):
        wid = lax.axis_index("s") * nc + lax.axis_index("c")
        base = wid * bpw

        cu = pltpu.async_copy(xu_hbm.at[pl.ds(base, bpw)], idx_u, sem)
        cm = pltpu.async_copy(xm_hbm.at[pl.ds(base, bpw)], idx_m, sem)
        cu.wait()
        cm.wait()

        for k in range(bpw // LANES):
            sl = pl.ds(k * LANES, LANES)
            eu = idx_u[sl] * WIDE
            em = idx_m[sl] * WIDE
            iu0[sl] = eu
            iu1[sl] = eu + 1
            im0[sl] = em
            im1[sl] = em + 1

        cps = []
        for j in range(nchunk):
            sl = pl.ds(j * CHUNK, CHUNK)
            cps.append(pltpu.async_copy(ut_hbm.at[iu0.at[sl]], u0v.at[sl], sem))
            cps.append(pltpu.async_copy(ut_hbm.at[iu1.at[sl]], u1v.at[sl], sem))
            cps.append(pltpu.async_copy(mt_hbm.at[im0.at[sl]], m0v.at[sl], sem))
            cps.append(pltpu.async_copy(mt_hbm.at[im1.at[sl]], m1v.at[sl], sem))
        for c in cps:
            c.wait()

        for k in range(bpw // LANES):
            sl = pl.ds(k * LANES, LANES)
            out_v[sl] = (_sigmoid(u0v[sl] - m0v[sl]) +
                         _sigmoid(u1v[sl] - m1v[sl])) * 0.5

        pltpu.sync_copy(out_v, out_hbm.at[pl.ds(base, bpw)])

    ut_wide = _widen(uid_table).reshape(-1)
    mt_wide = _widen(mid_table).reshape(-1)
    return sc_kernel(x_uid, x_mid, ut_wide, mt_wide)
